# Initial kernel scaffold; baseline (speedup 1.0000x reference)
import jax
import jax.numpy as jnp
from jax.experimental import pallas as pl


def kernel(entity_emb, edge_index, edge_type, weight, W1_w, W1_b, W2_w, W2_b):
    raise NotImplementedError("write your pallas kernel here")


# speedup vs baseline: 2.3144x; 2.3144x over previous
import functools

import jax
import jax.numpy as jnp
from jax import lax
from jax.experimental import pallas as pl
from jax.experimental.pallas import tpu as pltpu
from jax.experimental.pallas import tpu_sc as plsc

NC = 2
NS = 16
B = 64


def _make_agg(n, e, c, r):
    assert e % B == 0 and c % 16 == 0
    n_tiles = NC * NS
    n_pad = ((n + NS * B - 1) // (NS * B)) * (NS * B)
    rpt = n_pad // NS
    chunks = e // B
    full_trips = chunks // n_tiles
    extra = chunks % n_tiles

    mesh = plsc.VectorSubcoreMesh(core_axis_name="c", subcore_axis_name="s",
                                  num_cores=NC, num_subcores=NS)

    @functools.partial(
        pl.kernel,
        mesh=mesh,
        out_type=jax.ShapeDtypeStruct((NC, n_pad, c), jnp.float32),
        scratch_types=[
            pltpu.VMEM_SHARED((n_pad, c), jnp.float32),
            pltpu.VMEM((B,), jnp.int32),
            pltpu.VMEM((B,), jnp.int32),
            pltpu.VMEM((B,), jnp.int32),
            pltpu.VMEM((B, c), jnp.float32),
            pltpu.VMEM((B, c), jnp.float32),
            pltpu.SemaphoreType.DMA,
            pltpu.SemaphoreType.DMA,
        ],
    )
    def agg(res_hbm, head_hbm, tail_hbm, etype_hbm, w_hbm, sum_out,
            accum, tail_v, head_v, etype_v, rows_v, wrows_v, sem_r, sem_w):
        cid = lax.axis_index("c")
        sid = lax.axis_index("s")
        wid = sid * NC + cid
        base = sid * rpt

        zero16 = jnp.zeros((16,), jnp.float32)

        @pl.loop(0, B)
        def _(i):
            for k in range(c // 16):
                rows_v[i, pl.ds(k * 16, 16)] = zero16

        for q in range(rpt // B):
            pltpu.sync_copy(rows_v, accum.at[pl.ds(base + q * B, B)])
        plsc.subcore_barrier()

        ntrips = jnp.where(wid < extra, full_trips + 1, full_trips)

        @pl.loop(0, ntrips)
        def _(t):
            ebase = (wid + t * n_tiles) * B
            pltpu.sync_copy(tail_hbm.at[pl.ds(ebase, B)], tail_v)
            pltpu.sync_copy(etype_hbm.at[pl.ds(ebase, B)], etype_v)
            pltpu.sync_copy(head_hbm.at[pl.ds(ebase, B)], head_v)
            cp_r = pltpu.async_copy(res_hbm.at[tail_v], rows_v, sem_r)
            cp_w = pltpu.async_copy(w_hbm.at[etype_v], wrows_v, sem_w)
            cp_r.wait()
            cp_w.wait()

            @pl.loop(0, B)
            def _(ei):
                for k in range(c // 16):
                    sl = pl.ds(k * 16, 16)
                    rows_v[ei, sl] = rows_v[ei, sl] * wrows_v[ei, sl]

            pltpu.sync_copy(rows_v, accum.at[head_v], add=True)

        plsc.subcore_barrier()
        pltpu.sync_copy(accum.at[pl.ds(base, rpt)],
                        sum_out.at[cid, pl.ds(base, rpt)])

    return agg, n_pad


def _dense_body(ps_ref, res_ref, w1_ref, b1_ref, w2a_ref, w2b_ref,
                b2_ref, out_ref):
    s = ps_ref[0] + ps_ref[1]
    nrm = jnp.sqrt(jnp.sum(s * s, axis=1, keepdims=True))
    emb = s / jnp.maximum(nrm, 1e-12)
    res = res_ref[...]
    dot = functools.partial(jnp.dot, preferred_element_type=jnp.float32,
                            precision=lax.Precision.HIGHEST)
    x1 = dot(res + emb, w1_ref[...]) + b1_ref[...]
    x2 = dot(res, w2a_ref[...]) + dot(emb, w2b_ref[...]) + b2_ref[...]
    r1 = jnp.where(x1 >= 0, x1, 0.01 * x1)
    r2 = jnp.where(x2 >= 0, x2, 0.01 * x2)
    out_ref[...] = r1 + r2


def _dense_call(psum, res, w1, b1, w2a, w2b, b2, bm=512):
    n, c = res.shape
    n_pad = psum.shape[1]
    grid = (max(n, n_pad) + bm - 1) // bm
    full = lambda shape: pl.BlockSpec(shape, lambda i: (0,) * len(shape))
    return pl.pallas_call(
        _dense_body,
        grid=(grid,),
        in_specs=[
            pl.BlockSpec((NC, bm, c), lambda i: (0, i, 0)),
            pl.BlockSpec((bm, c), lambda i: (i, 0)),
            full((c, c)),
            full((1, c)),
            full((c, c)),
            full((c, c)),
            full((1, c)),
        ],
        out_specs=pl.BlockSpec((bm, c), lambda i: (i, 0)),
        out_shape=jax.ShapeDtypeStruct((n, c), jnp.float32),
    )(psum, res, w1, b1, w2a, w2b, b2)


def kernel(entity_emb, edge_index, edge_type, weight, W1_w, W1_b, W2_w, W2_b):
    n, c = entity_emb.shape
    e = edge_index.shape[1]
    r = weight.shape[0]
    n_hops = W1_w.shape[0]
    head = edge_index[0]
    tail = edge_index[1]
    etype = edge_type.astype(jnp.int32)

    agg, n_pad = _make_agg(n, e, c, r)

    res = entity_emb
    for i in range(n_hops):
        psum = agg(res, head, tail, etype, weight)
        res = _dense_call(psum, res,
                          W1_w[i], W1_b[i].reshape(1, c),
                          W2_w[i, :c], W2_w[i, c:], W2_b[i].reshape(1, c))
    return res, weight


# speedup vs baseline: 2.3378x; 1.0101x over previous
import functools

import jax
import jax.numpy as jnp
from jax import lax
from jax.experimental import pallas as pl
from jax.experimental.pallas import tpu as pltpu
from jax.experimental.pallas import tpu_sc as plsc

NC = 2
NS = 16
B = 128


def _make_agg(n, e, c, r):
    assert e % B == 0 and c % 16 == 0
    n_tiles = NC * NS
    n_pad = ((n + NS * B - 1) // (NS * B)) * (NS * B)
    rpt = n_pad // NS
    chunks = e // B
    full_trips = chunks // n_tiles
    extra = chunks % n_tiles

    mesh = plsc.VectorSubcoreMesh(core_axis_name="c", subcore_axis_name="s",
                                  num_cores=NC, num_subcores=NS)

    @functools.partial(
        pl.kernel,
        mesh=mesh,
        out_type=jax.ShapeDtypeStruct((NC, n_pad, c), jnp.float32),
        scratch_types=[
            pltpu.VMEM_SHARED((n_pad, c), jnp.float32),
            pltpu.VMEM((B,), jnp.int32),
            pltpu.VMEM((B,), jnp.int32),
            pltpu.VMEM((B,), jnp.int32),
            pltpu.VMEM((B, c), jnp.float32),
            pltpu.VMEM((B, c), jnp.float32),
            pltpu.SemaphoreType.DMA,
            pltpu.SemaphoreType.DMA,
        ],
    )
    def agg(res_hbm, head_hbm, tail_hbm, etype_hbm, w_hbm, sum_out,
            accum, tail_v, head_v, etype_v, rows_v, wrows_v, sem_r, sem_w):
        cid = lax.axis_index("c")
        sid = lax.axis_index("s")
        wid = sid * NC + cid
        base = sid * rpt

        zero16 = jnp.zeros((16,), jnp.float32)

        @pl.loop(0, B)
        def _(i):
            for k in range(c // 16):
                rows_v[i, pl.ds(k * 16, 16)] = zero16

        for q in range(rpt // B):
            pltpu.sync_copy(rows_v, accum.at[pl.ds(base + q * B, B)])
        plsc.subcore_barrier()

        ntrips = jnp.where(wid < extra, full_trips + 1, full_trips)

        @pl.loop(0, ntrips)
        def _(t):
            ebase = (wid + t * n_tiles) * B
            pltpu.sync_copy(tail_hbm.at[pl.ds(ebase, B)], tail_v)
            pltpu.sync_copy(etype_hbm.at[pl.ds(ebase, B)], etype_v)
            pltpu.sync_copy(head_hbm.at[pl.ds(ebase, B)], head_v)
            cp_r = pltpu.async_copy(res_hbm.at[tail_v], rows_v, sem_r)
            cp_w = pltpu.async_copy(w_hbm.at[etype_v], wrows_v, sem_w)
            cp_r.wait()
            cp_w.wait()

            @pl.loop(0, B)
            def _(ei):
                for k in range(c // 16):
                    sl = pl.ds(k * 16, 16)
                    rows_v[ei, sl] = rows_v[ei, sl] * wrows_v[ei, sl]

            pltpu.sync_copy(rows_v, accum.at[head_v], add=True)

        plsc.subcore_barrier()
        pltpu.sync_copy(accum.at[pl.ds(base, rpt)],
                        sum_out.at[cid, pl.ds(base, rpt)])

    return agg, n_pad


def _dense_body(ps_ref, res_ref, w1_ref, b1_ref, w2a_ref, w2b_ref,
                b2_ref, out_ref):
    s = ps_ref[0] + ps_ref[1]
    nrm = jnp.sqrt(jnp.sum(s * s, axis=1, keepdims=True))
    emb = s / jnp.maximum(nrm, 1e-12)
    res = res_ref[...]
    dot = functools.partial(jnp.dot, preferred_element_type=jnp.float32,
                            precision=lax.Precision.HIGHEST)
    x1 = dot(res + emb, w1_ref[...]) + b1_ref[...]
    x2 = dot(res, w2a_ref[...]) + dot(emb, w2b_ref[...]) + b2_ref[...]
    r1 = jnp.where(x1 >= 0, x1, 0.01 * x1)
    r2 = jnp.where(x2 >= 0, x2, 0.01 * x2)
    out_ref[...] = r1 + r2


def _dense_call(psum, res, w1, b1, w2a, w2b, b2, bm=512):
    n, c = res.shape
    n_pad = psum.shape[1]
    grid = (max(n, n_pad) + bm - 1) // bm
    full = lambda shape: pl.BlockSpec(shape, lambda i: (0,) * len(shape))
    return pl.pallas_call(
        _dense_body,
        grid=(grid,),
        in_specs=[
            pl.BlockSpec((NC, bm, c), lambda i: (0, i, 0)),
            pl.BlockSpec((bm, c), lambda i: (i, 0)),
            full((c, c)),
            full((1, c)),
            full((c, c)),
            full((c, c)),
            full((1, c)),
        ],
        out_specs=pl.BlockSpec((bm, c), lambda i: (i, 0)),
        out_shape=jax.ShapeDtypeStruct((n, c), jnp.float32),
    )(psum, res, w1, b1, w2a, w2b, b2)


def kernel(entity_emb, edge_index, edge_type, weight, W1_w, W1_b, W2_w, W2_b):
    n, c = entity_emb.shape
    e = edge_index.shape[1]
    r = weight.shape[0]
    n_hops = W1_w.shape[0]
    head = edge_index[0]
    tail = edge_index[1]
    etype = edge_type.astype(jnp.int32)

    agg, n_pad = _make_agg(n, e, c, r)

    res = entity_emb
    for i in range(n_hops):
        psum = agg(res, head, tail, etype, weight)
        res = _dense_call(psum, res,
                          W1_w[i], W1_b[i].reshape(1, c),
                          W2_w[i, :c], W2_w[i, c:], W2_b[i].reshape(1, c))
    return res, weight
